# Initial kernel scaffold; baseline (speedup 1.0000x reference)
import jax
import jax.numpy as jnp
from jax.experimental import pallas as pl


def kernel(inputs, embeddings):
    raise NotImplementedError("write your pallas kernel here")


# speedup vs baseline: 1.3060x; 1.3060x over previous
import functools

import jax
import jax.numpy as jnp
from jax import lax
from jax.experimental import pallas as pl
from jax.experimental.pallas import tpu as pltpu
from jax.experimental.pallas import tpu_sc as plsc

NC = 2
NS = 16
NW = NC * NS
C = 128
NBUF = 4


@functools.lru_cache(maxsize=None)
def _build(B: int, D: int):
    assert B % (NW * C) == 0
    nch = B // (NW * C)
    assert nch > NBUF and (nch - NBUF) % NBUF == 0
    mesh = plsc.VectorSubcoreMesh(core_axis_name="c", subcore_axis_name="s")

    @functools.partial(
        pl.kernel,
        mesh=mesh,
        out_type=jax.ShapeDtypeStruct((B // C, C, D), jnp.float32),
        scratch_types=[
            pltpu.VMEM((nch, C), jnp.int32),
            pltpu.VMEM((NBUF, C, D), jnp.float32),
        ] + [pltpu.SemaphoreType.DMA] * NBUF,
        compiler_params=pltpu.CompilerParams(use_tc_tiling_on_sc=False),
    )
    def gather_kernel(table_hbm, idx_hbm, out_hbm, idx_v, rows_v, *gsems):
        wid = lax.axis_index("s") * NC + lax.axis_index("c")
        pltpu.sync_copy(idx_hbm.at[wid], idx_v)
        chunk0 = wid * nch

        def start(j, b):
            pltpu.async_copy(table_hbm.at[idx_v.at[j]], rows_v.at[b], gsems[b])

        def finish(j, b):
            pltpu.make_async_copy(
                table_hbm.at[idx_v.at[j]], rows_v.at[b], gsems[b]
            ).wait()
            pltpu.sync_copy(rows_v.at[b], out_hbm.at[chunk0 + j])

        for b in range(NBUF):
            start(b, b)

        def body(m, carry):
            j0 = m * NBUF
            for b in range(NBUF):
                finish(j0 + b, b)
                start(j0 + b + NBUF, b)
            return carry

        lax.fori_loop(0, (nch - NBUF) // NBUF, body, 0)

        for b in range(NBUF):
            finish(nch - NBUF + b, b)

    return gather_kernel


def kernel(inputs, embeddings):
    B = inputs.shape[0] * inputs.shape[1]
    D = embeddings.shape[1]
    idx = inputs.astype(jnp.int32).reshape(NW, B // (NW * C), C)
    out = _build(B, D)(embeddings, idx)
    return out.reshape(inputs.shape[0], inputs.shape[1], D)


# speedup vs baseline: 1.3081x; 1.0016x over previous
import functools

import jax
import jax.numpy as jnp
from jax import lax
from jax.experimental import pallas as pl
from jax.experimental.pallas import tpu as pltpu
from jax.experimental.pallas import tpu_sc as plsc

NC = 2
NS = 16
NW = NC * NS
C = 128
NB = 8
P = 4


@functools.lru_cache(maxsize=None)
def _build(B: int, D: int):
    assert B % (NW * C) == 0
    nch = B // (NW * C)
    assert nch % NB == 0 and nch // NB >= 2
    mesh = plsc.VectorSubcoreMesh(core_axis_name="c", subcore_axis_name="s")

    @functools.partial(
        pl.kernel,
        mesh=mesh,
        out_type=jax.ShapeDtypeStruct((B // C, C, D), jnp.float32),
        scratch_types=[
            pltpu.VMEM((nch, C), jnp.int32),
            pltpu.VMEM((NB, C, D), jnp.float32),
        ] + [pltpu.SemaphoreType.DMA] * (2 * NB),
        compiler_params=pltpu.CompilerParams(use_tc_tiling_on_sc=False),
    )
    def gather_kernel(table_hbm, idx_hbm, out_hbm, idx_v, rows_v, *sems):
        gsems, wsems = sems[:NB], sems[NB:]
        wid = lax.axis_index("s") * NC + lax.axis_index("c")
        pltpu.sync_copy(idx_hbm.at[wid], idx_v)
        chunk0 = wid * nch

        def start_gather(j, b):
            pltpu.async_copy(table_hbm.at[idx_v.at[j]], rows_v.at[b], gsems[b])

        def wait_gather(j, b):
            pltpu.make_async_copy(
                table_hbm.at[idx_v.at[j]], rows_v.at[b], gsems[b]).wait()

        def start_write(j, b):
            pltpu.async_copy(rows_v.at[b], out_hbm.at[chunk0 + j], wsems[b])

        def wait_write(j, b):
            pltpu.make_async_copy(
                rows_v.at[b], out_hbm.at[chunk0 + j], wsems[b]).wait()

        def round8(j0, first, last):
            for b in range(NB):
                j = j0 + b
                wait_gather(j, b)
                if last and b >= NB - P:
                    pltpu.sync_copy(rows_v.at[b], out_hbm.at[chunk0 + j])
                else:
                    start_write(j, b)
                    bg = (b + P) % NB
                    if not (first and b < NB - P):
                        wait_write(j + P - NB, bg)
                    start_gather(j + P, bg)
            if last:
                for b in range(NB - P):
                    wait_write(j0 + b, b)

        for b in range(P):
            start_gather(b, b)
        round8(0, True, False)

        def body(m, carry):
            round8(m * NB, False, False)
            return carry

        lax.fori_loop(1, nch // NB - 1, body, 0)
        round8(nch - NB, False, True)

    return gather_kernel


def kernel(inputs, embeddings):
    B = inputs.shape[0] * inputs.shape[1]
    D = embeddings.shape[1]
    idx = inputs.astype(jnp.int32).reshape(NW, B // (NW * C), C)
    out = _build(B, D)(embeddings, idx)
    return out.reshape(inputs.shape[0], inputs.shape[1], D)


# speedup vs baseline: 1.3112x; 1.0024x over previous
import functools

import jax
import jax.numpy as jnp
from jax import lax
from jax.experimental import pallas as pl
from jax.experimental.pallas import tpu as pltpu
from jax.experimental.pallas import tpu_sc as plsc

NC = 2
NS = 16
NW = NC * NS
C = 128
NB = 10
P = 8


@functools.lru_cache(maxsize=None)
def _build(B: int, D: int):
    assert B % (NW * C) == 0
    nch = B // (NW * C)
    assert nch % NB == 0 and nch // NB >= 2
    mesh = plsc.VectorSubcoreMesh(core_axis_name="c", subcore_axis_name="s")

    @functools.partial(
        pl.kernel,
        mesh=mesh,
        out_type=jax.ShapeDtypeStruct((B // C, C, D), jnp.float32),
        scratch_types=[
            pltpu.VMEM((nch, C), jnp.int32),
            pltpu.VMEM((NB, C, D), jnp.float32),
        ] + [pltpu.SemaphoreType.DMA] * (2 * NB),
        compiler_params=pltpu.CompilerParams(use_tc_tiling_on_sc=False),
    )
    def gather_kernel(table_hbm, idx_hbm, out_hbm, idx_v, rows_v, *sems):
        gsems, wsems = sems[:NB], sems[NB:]
        wid = lax.axis_index("s") * NC + lax.axis_index("c")
        pltpu.sync_copy(idx_hbm.at[wid], idx_v)
        chunk0 = wid * nch

        def start_gather(j, b):
            pltpu.async_copy(table_hbm.at[idx_v.at[j]], rows_v.at[b], gsems[b])

        def wait_gather(j, b):
            pltpu.make_async_copy(
                table_hbm.at[idx_v.at[j]], rows_v.at[b], gsems[b]).wait()

        def start_write(j, b):
            pltpu.async_copy(rows_v.at[b], out_hbm.at[chunk0 + j], wsems[b])

        def wait_write(j, b):
            pltpu.make_async_copy(
                rows_v.at[b], out_hbm.at[chunk0 + j], wsems[b]).wait()

        def round8(j0, first, last):
            for b in range(NB):
                j = j0 + b
                wait_gather(j, b)
                if last and b >= NB - P:
                    pltpu.sync_copy(rows_v.at[b], out_hbm.at[chunk0 + j])
                else:
                    start_write(j, b)
                    bg = (b + P) % NB
                    if not (first and b < NB - P):
                        wait_write(j + P - NB, bg)
                    start_gather(j + P, bg)
            if last:
                for b in range(NB - P):
                    wait_write(j0 + b, b)

        for b in range(P):
            start_gather(b, b)
        round8(0, True, False)

        def body(m, carry):
            round8(m * NB, False, False)
            return carry

        lax.fori_loop(1, nch // NB - 1, body, 0)
        round8(nch - NB, False, True)

    return gather_kernel


def kernel(inputs, embeddings):
    B = inputs.shape[0] * inputs.shape[1]
    D = embeddings.shape[1]
    idx = inputs.astype(jnp.int32).reshape(NW, B // (NW * C), C)
    out = _build(B, D)(embeddings, idx)
    return out.reshape(inputs.shape[0], inputs.shape[1], D)
